# baseline (speedup 1.0000x reference)
import jax
import jax.numpy as jnp
from jax import lax
from jax.experimental import pallas as pl
from jax.experimental.pallas import tpu as pltpu
from jax.experimental.pallas import tpu_sc as plsc

DIM = 128
BATCH = 16384

NC = 2
NS = 16
L = 16
NW = NC * NS
BPW = BATCH // NW
CHUNK = 128
NCHUNK = BPW // CHUNK
GROUPS = CHUNK // L


def _gmf_body(user_hbm, item_hbm, utab_hbm, itab_hbm, out_hbm,
              u_idx, i_idx, u_buf, i_buf, o_v, accs, sem_u, sem_i):
    wid = lax.axis_index("s") * NC + lax.axis_index("c")
    base = wid * BPW

    pltpu.sync_copy(user_hbm.at[pl.ds(base, BPW)], u_idx)
    pltpu.sync_copy(item_hbm.at[pl.ds(base, BPW)], i_idx)

    iota = lax.broadcasted_iota(jnp.int32, (L,), 0)

    def issue(c):
        boff = (c % 2) * CHUNK
        pltpu.async_copy(utab_hbm.at[u_idx.at[pl.ds(c * CHUNK, CHUNK)]],
                         u_buf.at[pl.ds(boff, CHUNK)], sem_u)
        pltpu.async_copy(itab_hbm.at[i_idx.at[pl.ds(c * CHUNK, CHUNK)]],
                         i_buf.at[pl.ds(boff, CHUNK)], sem_i)

    issue(0)
    issue(1)

    def chunk_body(c, _):
        boff = (c % 2) * CHUNK
        pltpu.make_async_copy(utab_hbm.at[u_idx.at[pl.ds(0, CHUNK)]],
                              u_buf.at[pl.ds(boff, CHUNK)], sem_u).wait()
        pltpu.make_async_copy(itab_hbm.at[i_idx.at[pl.ds(0, CHUNK)]],
                              i_buf.at[pl.ds(boff, CHUNK)], sem_i).wait()

        def load_row(r):
            return ([u_buf[r, pl.ds(k * L, L)] for k in range(DIM // L)],
                    [i_buf[r, pl.ds(k * L, L)] for k in range(DIM // L)])

        def rows_phase(g):
            row = boff + g * L
            aoff = (g % 2) * L
            nxt_ld = load_row(row)
            for rr in range(L):
                us, vs = nxt_ld
                if rr + 1 < L:
                    nxt_ld = load_row(row + rr + 1)
                s0 = us[0] * vs[0]
                s1 = us[1] * vs[1]
                s0 = s0 + us[2] * vs[2]
                s1 = s1 + us[3] * vs[3]
                s0 = s0 + us[4] * vs[4]
                s1 = s1 + us[5] * vs[5]
                s0 = s0 + us[6] * vs[6]
                s1 = s1 + us[7] * vs[7]
                accs[aoff + rr, pl.ds(0, L)] = s0 + s1

        def reduce_phase(g):
            rvec = (g % 2) * L + iota
            t0 = plsc.load_gather(accs, [rvec, jnp.zeros((L,), jnp.int32)])
            t1 = plsc.load_gather(accs, [rvec, jnp.zeros((L,), jnp.int32) + 1])
            for j in range(2, L, 2):
                t0 = t0 + plsc.load_gather(
                    accs, [rvec, jnp.zeros((L,), jnp.int32) + j])
                t1 = t1 + plsc.load_gather(
                    accs, [rvec, jnp.zeros((L,), jnp.int32) + j + 1])
            tot = t0 + t1
            o_v[pl.ds(c * CHUNK + g * L, L)] = 1.0 / (1.0 + jnp.exp(-tot))

        rows_phase(0)

        def group_body(t, _):
            reduce_phase(t)
            rows_phase(t + 1)
            return 0

        lax.fori_loop(0, GROUPS - 1, group_body, 0)
        reduce_phase(GROUPS - 1)

        @pl.when(c + 2 < NCHUNK)
        def _():
            pltpu.async_copy(
                utab_hbm.at[u_idx.at[pl.ds((c + 2) * CHUNK, CHUNK)]],
                u_buf.at[pl.ds(boff, CHUNK)], sem_u)
            pltpu.async_copy(
                itab_hbm.at[i_idx.at[pl.ds((c + 2) * CHUNK, CHUNK)]],
                i_buf.at[pl.ds(boff, CHUNK)], sem_i)

        return 0

    lax.fori_loop(0, NCHUNK, chunk_body, 0)

    pltpu.sync_copy(o_v, out_hbm.at[pl.ds(base, BPW)])


@jax.jit
def _gmf(user1d, item1d, user_table, item_table):
    mesh = plsc.VectorSubcoreMesh(core_axis_name="c", subcore_axis_name="s")
    kern = pl.kernel(
        _gmf_body,
        mesh=mesh,
        out_type=jax.ShapeDtypeStruct((BATCH,), jnp.float32),
        compiler_params=pltpu.CompilerParams(needs_layout_passes=False),
        scratch_types=[
            pltpu.VMEM((BPW,), jnp.int32),
            pltpu.VMEM((BPW,), jnp.int32),
            pltpu.VMEM((2 * CHUNK, DIM), jnp.float32),
            pltpu.VMEM((2 * CHUNK, DIM), jnp.float32),
            pltpu.VMEM((BPW,), jnp.float32),
            pltpu.VMEM((2 * L, L + 1), jnp.float32),
            pltpu.SemaphoreType.DMA,
            pltpu.SemaphoreType.DMA,
        ],
    )
    return kern(user1d, item1d, user_table, item_table)


def kernel(user, item, user_table, item_table):
    return _gmf(user.astype(jnp.int32), item.astype(jnp.int32),
                user_table, item_table)


# speedup vs baseline: 1.0284x; 1.0284x over previous
import jax
import jax.numpy as jnp
from jax import lax
from jax.experimental import pallas as pl
from jax.experimental.pallas import tpu as pltpu
from jax.experimental.pallas import tpu_sc as plsc

DIM = 128
BATCH = 16384

NC = 2
NS = 16
L = 16
NW = NC * NS
BPW = BATCH // NW
CHUNK = 128
NCHUNK = BPW // CHUNK
GROUPS = CHUNK // L


def _gmf_body(user_hbm, item_hbm, utab_hbm, itab_hbm, out_hbm,
              u_idx, i_idx, u_buf, i_buf, o_v, accs, sem_u, sem_i):
    wid = lax.axis_index("s") * NC + lax.axis_index("c")
    base = wid * BPW

    cu = pltpu.async_copy(user_hbm.at[pl.ds(base, BPW)], u_idx, sem_u)
    ci = pltpu.async_copy(item_hbm.at[pl.ds(base, BPW)], i_idx, sem_i)
    cu.wait()
    ci.wait()

    iota = lax.broadcasted_iota(jnp.int32, (L,), 0)

    def issue(c):
        boff = (c % 2) * CHUNK
        pltpu.async_copy(utab_hbm.at[u_idx.at[pl.ds(c * CHUNK, CHUNK)]],
                         u_buf.at[pl.ds(boff, CHUNK)], sem_u)
        pltpu.async_copy(itab_hbm.at[i_idx.at[pl.ds(c * CHUNK, CHUNK)]],
                         i_buf.at[pl.ds(boff, CHUNK)], sem_i)

    issue(0)
    issue(1)

    def chunk_body(c, _):
        boff = (c % 2) * CHUNK
        pltpu.make_async_copy(utab_hbm.at[u_idx.at[pl.ds(0, CHUNK)]],
                              u_buf.at[pl.ds(boff, CHUNK)], sem_u).wait()
        pltpu.make_async_copy(itab_hbm.at[i_idx.at[pl.ds(0, CHUNK)]],
                              i_buf.at[pl.ds(boff, CHUNK)], sem_i).wait()

        def load_row(r):
            return ([u_buf[r, pl.ds(k * L, L)] for k in range(DIM // L)],
                    [i_buf[r, pl.ds(k * L, L)] for k in range(DIM // L)])

        def rows_phase(g):
            row = boff + g * L
            aoff = (g % 2) * L
            nxt_ld = load_row(row)
            for rr in range(L):
                us, vs = nxt_ld
                if rr + 1 < L:
                    nxt_ld = load_row(row + rr + 1)
                s0 = us[0] * vs[0]
                s1 = us[1] * vs[1]
                s0 = s0 + us[2] * vs[2]
                s1 = s1 + us[3] * vs[3]
                s0 = s0 + us[4] * vs[4]
                s1 = s1 + us[5] * vs[5]
                s0 = s0 + us[6] * vs[6]
                s1 = s1 + us[7] * vs[7]
                accs[aoff + rr, pl.ds(0, L)] = s0 + s1

        def reduce_phase(g):
            rvec = (g % 2) * L + iota
            t0 = plsc.load_gather(accs, [rvec, jnp.zeros((L,), jnp.int32)])
            t1 = plsc.load_gather(accs, [rvec, jnp.zeros((L,), jnp.int32) + 1])
            for j in range(2, L, 2):
                t0 = t0 + plsc.load_gather(
                    accs, [rvec, jnp.zeros((L,), jnp.int32) + j])
                t1 = t1 + plsc.load_gather(
                    accs, [rvec, jnp.zeros((L,), jnp.int32) + j + 1])
            tot = t0 + t1
            o_v[pl.ds(c * CHUNK + g * L, L)] = 1.0 / (1.0 + jnp.exp(-tot))

        rows_phase(0)

        def group_body(t, _):
            reduce_phase(t)
            rows_phase(t + 1)
            return 0

        lax.fori_loop(0, GROUPS - 1, group_body, 0)
        reduce_phase(GROUPS - 1)

        @pl.when(c + 2 < NCHUNK)
        def _():
            pltpu.async_copy(
                utab_hbm.at[u_idx.at[pl.ds((c + 2) * CHUNK, CHUNK)]],
                u_buf.at[pl.ds(boff, CHUNK)], sem_u)
            pltpu.async_copy(
                itab_hbm.at[i_idx.at[pl.ds((c + 2) * CHUNK, CHUNK)]],
                i_buf.at[pl.ds(boff, CHUNK)], sem_i)

        return 0

    lax.fori_loop(0, NCHUNK, chunk_body, 0)

    pltpu.sync_copy(o_v, out_hbm.at[pl.ds(base, BPW)])


@jax.jit
def _gmf(user1d, item1d, user_table, item_table):
    mesh = plsc.VectorSubcoreMesh(core_axis_name="c", subcore_axis_name="s")
    kern = pl.kernel(
        _gmf_body,
        mesh=mesh,
        out_type=jax.ShapeDtypeStruct((BATCH,), jnp.float32),
        compiler_params=pltpu.CompilerParams(needs_layout_passes=False),
        scratch_types=[
            pltpu.VMEM((BPW,), jnp.int32),
            pltpu.VMEM((BPW,), jnp.int32),
            pltpu.VMEM((2 * CHUNK, DIM), jnp.float32),
            pltpu.VMEM((2 * CHUNK, DIM), jnp.float32),
            pltpu.VMEM((BPW,), jnp.float32),
            pltpu.VMEM((2 * L, L + 1), jnp.float32),
            pltpu.SemaphoreType.DMA,
            pltpu.SemaphoreType.DMA,
        ],
    )
    return kern(user1d, item1d, user_table, item_table)


def kernel(user, item, user_table, item_table):
    return _gmf(user.astype(jnp.int32), item.astype(jnp.int32),
                user_table, item_table)


# speedup vs baseline: 1.0336x; 1.0051x over previous
import jax
import jax.numpy as jnp
from jax import lax
from jax.experimental import pallas as pl
from jax.experimental.pallas import tpu as pltpu
from jax.experimental.pallas import tpu_sc as plsc

DIM = 128
BATCH = 16384

NC = 2
NS = 16
L = 16
NW = NC * NS
BPW = BATCH // NW
CHUNK = 128
NCHUNK = BPW // CHUNK
GROUPS = CHUNK // L


def _gmf_body(user_hbm, item_hbm, utab_hbm, itab_hbm, out_hbm,
              u_idx, i_idx, u_buf, i_buf, o_v, accs, sem_u, sem_i):
    wid = lax.axis_index("s") * NC + lax.axis_index("c")
    base = wid * BPW

    cu = pltpu.async_copy(user_hbm.at[pl.ds(base, BPW)], u_idx, sem_u)
    ci = pltpu.async_copy(item_hbm.at[pl.ds(base, BPW)], i_idx, sem_i)
    cu.wait()
    ci.wait()

    iota = lax.broadcasted_iota(jnp.int32, (L,), 0)

    def issue(c):
        boff = (c % 3) * CHUNK
        pltpu.async_copy(utab_hbm.at[u_idx.at[pl.ds(c * CHUNK, CHUNK)]],
                         u_buf.at[pl.ds(boff, CHUNK)], sem_u)
        pltpu.async_copy(itab_hbm.at[i_idx.at[pl.ds(c * CHUNK, CHUNK)]],
                         i_buf.at[pl.ds(boff, CHUNK)], sem_i)

    issue(0)
    issue(1)
    issue(2)

    def chunk_body(c, _):
        boff = (c % 3) * CHUNK
        pltpu.make_async_copy(utab_hbm.at[u_idx.at[pl.ds(0, CHUNK)]],
                              u_buf.at[pl.ds(boff, CHUNK)], sem_u).wait()
        pltpu.make_async_copy(itab_hbm.at[i_idx.at[pl.ds(0, CHUNK)]],
                              i_buf.at[pl.ds(boff, CHUNK)], sem_i).wait()

        def load_row(r):
            return ([u_buf[r, pl.ds(k * L, L)] for k in range(DIM // L)],
                    [i_buf[r, pl.ds(k * L, L)] for k in range(DIM // L)])

        def rows_phase(g):
            row = boff + g * L
            aoff = (g % 2) * L
            nxt_ld = load_row(row)
            for rr in range(L):
                us, vs = nxt_ld
                if rr + 1 < L:
                    nxt_ld = load_row(row + rr + 1)
                s0 = us[0] * vs[0]
                s1 = us[1] * vs[1]
                s0 = s0 + us[2] * vs[2]
                s1 = s1 + us[3] * vs[3]
                s0 = s0 + us[4] * vs[4]
                s1 = s1 + us[5] * vs[5]
                s0 = s0 + us[6] * vs[6]
                s1 = s1 + us[7] * vs[7]
                accs[aoff + rr, pl.ds(0, L)] = s0 + s1

        def reduce_phase(g):
            rvec = (g % 2) * L + iota
            t0 = plsc.load_gather(accs, [rvec, jnp.zeros((L,), jnp.int32)])
            t1 = plsc.load_gather(accs, [rvec, jnp.zeros((L,), jnp.int32) + 1])
            for j in range(2, L, 2):
                t0 = t0 + plsc.load_gather(
                    accs, [rvec, jnp.zeros((L,), jnp.int32) + j])
                t1 = t1 + plsc.load_gather(
                    accs, [rvec, jnp.zeros((L,), jnp.int32) + j + 1])
            tot = t0 + t1
            o_v[pl.ds(c * CHUNK + g * L, L)] = 1.0 / (1.0 + jnp.exp(-tot))

        rows_phase(0)

        def group_body(t, _):
            reduce_phase(t)
            rows_phase(t + 1)
            return 0

        lax.fori_loop(0, GROUPS - 1, group_body, 0)
        reduce_phase(GROUPS - 1)

        @pl.when(c + 3 < NCHUNK)
        def _():
            pltpu.async_copy(
                utab_hbm.at[u_idx.at[pl.ds((c + 3) * CHUNK, CHUNK)]],
                u_buf.at[pl.ds(boff, CHUNK)], sem_u)
            pltpu.async_copy(
                itab_hbm.at[i_idx.at[pl.ds((c + 3) * CHUNK, CHUNK)]],
                i_buf.at[pl.ds(boff, CHUNK)], sem_i)

        return 0

    lax.fori_loop(0, NCHUNK, chunk_body, 0)

    pltpu.sync_copy(o_v, out_hbm.at[pl.ds(base, BPW)])


@jax.jit
def _gmf(user1d, item1d, user_table, item_table):
    mesh = plsc.VectorSubcoreMesh(core_axis_name="c", subcore_axis_name="s")
    kern = pl.kernel(
        _gmf_body,
        mesh=mesh,
        out_type=jax.ShapeDtypeStruct((BATCH,), jnp.float32),
        compiler_params=pltpu.CompilerParams(needs_layout_passes=False),
        scratch_types=[
            pltpu.VMEM((BPW,), jnp.int32),
            pltpu.VMEM((BPW,), jnp.int32),
            pltpu.VMEM((3 * CHUNK, DIM), jnp.float32),
            pltpu.VMEM((3 * CHUNK, DIM), jnp.float32),
            pltpu.VMEM((BPW,), jnp.float32),
            pltpu.VMEM((2 * L, L + 1), jnp.float32),
            pltpu.SemaphoreType.DMA,
            pltpu.SemaphoreType.DMA,
        ],
    )
    return kern(user1d, item1d, user_table, item_table)


def kernel(user, item, user_table, item_table):
    return _gmf(user.astype(jnp.int32), item.astype(jnp.int32),
                user_table, item_table)
